# baseline (speedup 1.0000x reference)
import functools

import jax
import jax.numpy as jnp
import numpy as np
from jax import lax
from jax.experimental import pallas as pl
from jax.experimental.pallas import tpu as pltpu
from jax.experimental.pallas import tpu_sc as plsc

C = 8
T = 262144
N = 2097152

STEP = float(np.float32(1.0) / np.float32(T - 1))
SCALE = float(np.float32(T - 1))

NC, NS, L = 2, 16, 16
NW = NC * NS
CHUNK = 2048
GROUP = 128
NGROUP = CHUNK // GROUP
NVPG = GROUP // L
QPW = N // NW
NCHUNK = QPW // CHUNK


BB = 2048
NB = T // BB


def _build_body(us_ref, prev_ref, out_ref):
    ub = us_ref[...]
    pid = pl.program_id(0)
    pall = prev_ref[...]
    sel = lax.broadcasted_iota(jnp.int32, (1, NB), 1) == pid
    pc = jnp.sum(jnp.where(sel, pall, jnp.float32(0.0)), axis=1,
                 keepdims=True)
    ubp = jnp.concatenate([pc, ub[:, :-1]], axis=1)
    jint = lax.broadcasted_iota(jnp.int32, (1, BB), 1) + pid * BB
    jf = jint.astype(jnp.float32)
    step = jnp.float32(STEP)
    dx = jf * step - (jf - 1.0) * step
    d = (ub - ubp) / dx
    d = jnp.where(jint >= 2, d, jnp.float32(0.0))
    x = jnp.concatenate([ub, d], axis=0)
    xt = x.T.reshape(BB // 8, 8, 2 * C)
    for r in range(8):
        out_ref[:, pl.ds(16 * r, 16)] = xt[:, r, :]


def _build_table(us, prev_col):
    return pl.pallas_call(
        _build_body,
        grid=(NB,),
        in_specs=[
            pl.BlockSpec((C, BB), lambda i: (0, i)),
            pl.BlockSpec((C, NB), lambda i: (0, 0)),
        ],
        out_specs=pl.BlockSpec((BB // 8, 128), lambda i: (i, 0)),
        out_shape=jax.ShapeDtypeStruct((T // 8, 128), jnp.float32),
    )(us, prev_col)


def _sc_body(p_hbm, t_hbm, out_hbm,
             t_v0, t_v1, off_v0, off_v1, idx_v0, idx_v1,
             rows_v0, rows_v1, out_v0, out_v1,
             sem_t, sem_g0, sem_g1):
    wid = lax.axis_index("s") * NC + lax.axis_index("c")
    w_base = wid * QPW
    step = jnp.float32(STEP)
    scale = jnp.float32(SCALE)

    t_bufs = (t_v0, t_v1)
    off_bufs = (off_v0, off_v1)
    idx_bufs = (idx_v0, idx_v1)
    rows_bufs = (rows_v0, rows_v1)
    out_bufs = (out_v0, out_v1)
    sem_g = (sem_g0, sem_g1)

    def phase1(t_v, off_v, idx_v):
        for g in range(NGROUP):
            def vbody(i, carry2, g=g):
                q0 = g * GROUP + i * L
                t16 = t_v[pl.ds(q0, L)]
                f = t16 * scale
                j = f.astype(jnp.int32)
                jf = j.astype(jnp.float32)
                one = jnp.full((L,), 1, jnp.int32)
                zero = jnp.full((L,), 0, jnp.int32)
                a = jnp.where(jf * step < t16, one, zero)
                b = jnp.where((jf + 1.0) * step < t16, one, zero)
                idx = j + a + b - 1
                off_v[pl.ds(q0, L)] = t16 - idx.astype(jnp.float32) * step
                idx_v[g, pl.ds(i * L, L)] = idx
                return carry2

            lax.fori_loop(0, NVPG, vbody, 0, unroll=4)

    def fire_gathers(b):
        for g in range(NGROUP):
            pltpu.async_copy(
                p_hbm.at[idx_bufs[b].at[g]],
                rows_bufs[b].at[pl.ds(g * GROUP, GROUP)],
                sem_g[b],
            )

    def drain_gathers(b):
        for g in range(NGROUP):
            pltpu.make_async_copy(
                p_hbm.at[idx_bufs[b].at[g]],
                rows_bufs[b].at[pl.ds(g * GROUP, GROUP)],
                sem_g[b],
            ).wait()

    def phase3_and_write(k, b):
        rows_v, off_v, out_v = rows_bufs[b], off_bufs[b], out_bufs[b]

        def obody(jj, carry2):
            def sbody(s, carry3):
                q0 = jj * 128 + s * L
                off16 = off_v[pl.ds(q0, L)]
                qi = q0 + lax.iota(jnp.int32, L)
                for c in range(C):
                    ci = jnp.full((L,), c, jnp.int32)
                    di = jnp.full((L,), c + C, jnp.int32)
                    usc = plsc.load_gather(rows_v, [qi, ci])
                    dc = plsc.load_gather(rows_v, [qi, di])
                    out_v[jj, c, pl.ds(s * L, L)] = usc + dc * off16
                return carry3

            return lax.fori_loop(0, 128 // L, sbody, carry2, unroll=2)

        lax.fori_loop(0, CHUNK // 128, obody, 0)
        base = w_base + k * CHUNK
        pltpu.sync_copy(out_v, out_hbm.at[pl.ds(base // 128, CHUNK // 128)])

    def start_tload(k, b):
        pltpu.async_copy(
            t_hbm.at[pl.ds(w_base + k * CHUNK, CHUNK)], t_bufs[b], sem_t)

    def wait_tload(b):
        pltpu.make_async_copy(
            t_hbm.at[pl.ds(w_base, CHUNK)], t_bufs[b], sem_t).wait()

    def half(kk, parity):
        k_new = 2 * kk + (1 if parity == 0 else 2)
        bn = 1 if parity == 0 else 0
        br = 1 - bn

        @pl.when(k_new < NCHUNK)
        def _():
            wait_tload(bn)
            phase1(t_bufs[bn], off_bufs[bn], idx_bufs[bn])
            fire_gathers(bn)

        @pl.when(k_new + 1 < NCHUNK)
        def _():
            start_tload(k_new + 1, br)

        drain_gathers(br)
        phase3_and_write(k_new - 1, br)

    pltpu.sync_copy(t_hbm.at[pl.ds(w_base, CHUNK)], t_v0)
    phase1(t_v0, off_v0, idx_v0)
    fire_gathers(0)
    start_tload(1, 1)

    def pair_body(kk, carry):
        half(kk, 0)
        half(kk, 1)
        return carry

    lax.fori_loop(0, NCHUNK // 2, pair_body, 0)


def _sc_lookup(p_table, t):
    mesh = plsc.VectorSubcoreMesh(core_axis_name="c", subcore_axis_name="s")
    fn = functools.partial(
        pl.kernel,
        mesh=mesh,
        out_type=jax.ShapeDtypeStruct((N // 128, C, 128), jnp.float32),
        scratch_types=[
            pltpu.VMEM((CHUNK,), jnp.float32),
            pltpu.VMEM((CHUNK,), jnp.float32),
            pltpu.VMEM((CHUNK,), jnp.float32),
            pltpu.VMEM((CHUNK,), jnp.float32),
            pltpu.VMEM((NGROUP, GROUP), jnp.int32),
            pltpu.VMEM((NGROUP, GROUP), jnp.int32),
            pltpu.VMEM((CHUNK, 2 * C), jnp.float32),
            pltpu.VMEM((CHUNK, 2 * C), jnp.float32),
            pltpu.VMEM((CHUNK // 128, C, 128), jnp.float32),
            pltpu.VMEM((CHUNK // 128, C, 128), jnp.float32),
            pltpu.SemaphoreType.DMA,
            pltpu.SemaphoreType.DMA,
            pltpu.SemaphoreType.DMA,
        ],
        compiler_params=pltpu.CompilerParams(
            use_tc_tiling_on_sc=False, needs_layout_passes=False),
    )(_sc_body)
    return fn(p_table, t)


KB = 128


def _format_body(a_ref, out_ref):
    for l in range(KB):
        out_ref[:, pl.ds(l * 128, 128)] = a_ref[l]


def _format(a):
    return pl.pallas_call(
        _format_body,
        grid=(N // (128 * KB),),
        in_specs=[pl.BlockSpec((KB, C, 128), lambda i: (i, 0, 0))],
        out_specs=pl.BlockSpec((C, KB * 128), lambda i: (0, i)),
        out_shape=jax.ShapeDtypeStruct((C, N), jnp.float32),
    )(a)


def kernel(x, t, us, t_range):
    del x, t_range
    prev_col = jnp.concatenate(
        [us[:, :1], us[:, BB - 1 : T - 1 : BB]], axis=1)
    p_table = _build_table(us, prev_col).reshape(T, 2 * C)
    return _format(_sc_lookup(p_table, t))


# speedup vs baseline: 1.0163x; 1.0163x over previous
import functools

import jax
import jax.numpy as jnp
import numpy as np
from jax import lax
from jax.experimental import pallas as pl
from jax.experimental.pallas import tpu as pltpu
from jax.experimental.pallas import tpu_sc as plsc

C = 8
T = 262144
N = 2097152

STEP = float(np.float32(1.0) / np.float32(T - 1))
SCALE = float(np.float32(T - 1))

NC, NS, L = 2, 16, 16
NW = NC * NS
CHUNK = 1024
GROUP = 128
NGROUP = CHUNK // GROUP
NVPG = GROUP // L
QPW = N // NW
NCHUNK = QPW // CHUNK


BB = 2048
NB = T // BB


def _build_body(us_ref, prev_ref, out_ref):
    ub = us_ref[...]
    pid = pl.program_id(0)
    pall = prev_ref[...]
    sel = lax.broadcasted_iota(jnp.int32, (1, NB), 1) == pid
    pc = jnp.sum(jnp.where(sel, pall, jnp.float32(0.0)), axis=1,
                 keepdims=True)
    ubp = jnp.concatenate([pc, ub[:, :-1]], axis=1)
    jint = lax.broadcasted_iota(jnp.int32, (1, BB), 1) + pid * BB
    jf = jint.astype(jnp.float32)
    step = jnp.float32(STEP)
    dx = jf * step - (jf - 1.0) * step
    d = (ub - ubp) / dx
    d = jnp.where(jint >= 2, d, jnp.float32(0.0))
    x = jnp.concatenate([ub, d], axis=0)
    xt = x.T.reshape(BB // 8, 8, 2 * C)
    for r in range(8):
        out_ref[:, pl.ds(16 * r, 16)] = xt[:, r, :]


def _build_table(us, prev_col):
    return pl.pallas_call(
        _build_body,
        grid=(NB,),
        in_specs=[
            pl.BlockSpec((C, BB), lambda i: (0, i)),
            pl.BlockSpec((C, NB), lambda i: (0, 0)),
        ],
        out_specs=pl.BlockSpec((BB // 8, 128), lambda i: (i, 0)),
        out_shape=jax.ShapeDtypeStruct((T // 8, 128), jnp.float32),
    )(us, prev_col)


def _sc_body(p_hbm, t_hbm, out_hbm,
             t_v0, t_v1, off_v0, off_v1, idx_v0, idx_v1,
             rows_v0, rows_v1, out_v0, out_v1,
             sem_t, sem_g0, sem_g1):
    wid = lax.axis_index("s") * NC + lax.axis_index("c")
    w_base = wid * QPW
    step = jnp.float32(STEP)
    scale = jnp.float32(SCALE)

    t_bufs = (t_v0, t_v1)
    off_bufs = (off_v0, off_v1)
    idx_bufs = (idx_v0, idx_v1)
    rows_bufs = (rows_v0, rows_v1)
    out_bufs = (out_v0, out_v1)
    sem_g = (sem_g0, sem_g1)

    def phase1(t_v, off_v, idx_v):
        for g in range(NGROUP):
            def vbody(i, carry2, g=g):
                q0 = g * GROUP + i * L
                t16 = t_v[pl.ds(q0, L)]
                f = t16 * scale
                j = f.astype(jnp.int32)
                jf = j.astype(jnp.float32)
                one = jnp.full((L,), 1, jnp.int32)
                zero = jnp.full((L,), 0, jnp.int32)
                a = jnp.where(jf * step < t16, one, zero)
                b = jnp.where((jf + 1.0) * step < t16, one, zero)
                idx = j + a + b - 1
                off_v[pl.ds(q0, L)] = t16 - idx.astype(jnp.float32) * step
                idx_v[g, pl.ds(i * L, L)] = idx
                return carry2

            lax.fori_loop(0, NVPG, vbody, 0, unroll=4)

    def fire_gathers(b):
        for g in range(NGROUP):
            pltpu.async_copy(
                p_hbm.at[idx_bufs[b].at[g]],
                rows_bufs[b].at[pl.ds(g * GROUP, GROUP)],
                sem_g[b],
            )

    def drain_gathers(b):
        for g in range(NGROUP):
            pltpu.make_async_copy(
                p_hbm.at[idx_bufs[b].at[g]],
                rows_bufs[b].at[pl.ds(g * GROUP, GROUP)],
                sem_g[b],
            ).wait()

    def phase3_and_write(k, b):
        rows_v, off_v, out_v = rows_bufs[b], off_bufs[b], out_bufs[b]

        def obody(jj, carry2):
            def sbody(s, carry3):
                q0 = jj * 128 + s * L
                off16 = off_v[pl.ds(q0, L)]
                qi = q0 + lax.iota(jnp.int32, L)
                for c in range(C):
                    ci = jnp.full((L,), c, jnp.int32)
                    di = jnp.full((L,), c + C, jnp.int32)
                    usc = plsc.load_gather(rows_v, [qi, ci])
                    dc = plsc.load_gather(rows_v, [qi, di])
                    out_v[jj, c, pl.ds(s * L, L)] = usc + dc * off16
                return carry3

            return lax.fori_loop(0, 128 // L, sbody, carry2, unroll=2)

        lax.fori_loop(0, CHUNK // 128, obody, 0)
        base = w_base + k * CHUNK
        pltpu.sync_copy(out_v, out_hbm.at[pl.ds(base // 128, CHUNK // 128)])

    def start_tload(k, b):
        pltpu.async_copy(
            t_hbm.at[pl.ds(w_base + k * CHUNK, CHUNK)], t_bufs[b], sem_t)

    def wait_tload(b):
        pltpu.make_async_copy(
            t_hbm.at[pl.ds(w_base, CHUNK)], t_bufs[b], sem_t).wait()

    def half(kk, parity):
        k_new = 2 * kk + (1 if parity == 0 else 2)
        bn = 1 if parity == 0 else 0
        br = 1 - bn

        @pl.when(k_new < NCHUNK)
        def _():
            wait_tload(bn)
            phase1(t_bufs[bn], off_bufs[bn], idx_bufs[bn])
            fire_gathers(bn)

        @pl.when(k_new + 1 < NCHUNK)
        def _():
            start_tload(k_new + 1, br)

        drain_gathers(br)
        phase3_and_write(k_new - 1, br)

    pltpu.sync_copy(t_hbm.at[pl.ds(w_base, CHUNK)], t_v0)
    phase1(t_v0, off_v0, idx_v0)
    fire_gathers(0)
    start_tload(1, 1)

    def pair_body(kk, carry):
        half(kk, 0)
        half(kk, 1)
        return carry

    lax.fori_loop(0, NCHUNK // 2, pair_body, 0)


def _sc_lookup(p_table, t):
    mesh = plsc.VectorSubcoreMesh(core_axis_name="c", subcore_axis_name="s")
    fn = functools.partial(
        pl.kernel,
        mesh=mesh,
        out_type=jax.ShapeDtypeStruct((N // 128, C, 128), jnp.float32),
        scratch_types=[
            pltpu.VMEM((CHUNK,), jnp.float32),
            pltpu.VMEM((CHUNK,), jnp.float32),
            pltpu.VMEM((CHUNK,), jnp.float32),
            pltpu.VMEM((CHUNK,), jnp.float32),
            pltpu.VMEM((NGROUP, GROUP), jnp.int32),
            pltpu.VMEM((NGROUP, GROUP), jnp.int32),
            pltpu.VMEM((CHUNK, 2 * C), jnp.float32),
            pltpu.VMEM((CHUNK, 2 * C), jnp.float32),
            pltpu.VMEM((CHUNK // 128, C, 128), jnp.float32),
            pltpu.VMEM((CHUNK // 128, C, 128), jnp.float32),
            pltpu.SemaphoreType.DMA,
            pltpu.SemaphoreType.DMA,
            pltpu.SemaphoreType.DMA,
        ],
        compiler_params=pltpu.CompilerParams(
            use_tc_tiling_on_sc=False, needs_layout_passes=False),
    )(_sc_body)
    return fn(p_table, t)


KB = 128


def _format_body(a_ref, out_ref):
    for l in range(KB):
        out_ref[:, pl.ds(l * 128, 128)] = a_ref[l]


def _format(a):
    return pl.pallas_call(
        _format_body,
        grid=(N // (128 * KB),),
        in_specs=[pl.BlockSpec((KB, C, 128), lambda i: (i, 0, 0))],
        out_specs=pl.BlockSpec((C, KB * 128), lambda i: (0, i)),
        out_shape=jax.ShapeDtypeStruct((C, N), jnp.float32),
    )(a)


def kernel(x, t, us, t_range):
    del x, t_range
    prev_col = jnp.concatenate(
        [us[:, :1], us[:, BB - 1 : T - 1 : BB]], axis=1)
    p_table = _build_table(us, prev_col).reshape(T, 2 * C)
    return _format(_sc_lookup(p_table, t))


# speedup vs baseline: 1.0694x; 1.0523x over previous
import functools

import jax
import jax.numpy as jnp
import numpy as np
from jax import lax
from jax.experimental import pallas as pl
from jax.experimental.pallas import tpu as pltpu
from jax.experimental.pallas import tpu_sc as plsc

C = 8
T = 262144
N = 2097152

STEP = float(np.float32(1.0) / np.float32(T - 1))
SCALE = float(np.float32(T - 1))

RW = 24
NC, NS, L = 2, 16, 16
NW = NC * NS
CHUNK = 1024
GROUP = 128
NGROUP = CHUNK // GROUP
NVPG = GROUP // L
QPW = N // NW
NCHUNK = QPW // CHUNK


BB = 2048
NB = T // BB


def _build_body(us_ref, prev_ref, out_ref):
    ub = us_ref[...]
    pid = pl.program_id(0)
    pall = prev_ref[...]
    sel = lax.broadcasted_iota(jnp.int32, (1, NB), 1) == pid
    pc = jnp.sum(jnp.where(sel, pall, jnp.float32(0.0)), axis=1,
                 keepdims=True)
    ubp = jnp.concatenate([pc, ub[:, :-1]], axis=1)
    jint = lax.broadcasted_iota(jnp.int32, (1, BB), 1) + pid * BB
    jf = jint.astype(jnp.float32)
    step = jnp.float32(STEP)
    dx = jf * step - (jf - 1.0) * step
    d = (ub - ubp) / dx
    d = jnp.where(jint >= 2, d, jnp.float32(0.0))
    x = jnp.concatenate([ub, d], axis=0)
    xt = x.T.reshape(BB // 16, 16, 2 * C)
    z8 = jnp.zeros((BB // 16, 8), jnp.float32)

    def rowc(r):
        return jnp.concatenate([xt[:, r, :], z8], axis=1)

    p0 = jnp.concatenate(
        [rowc(0), rowc(1), rowc(2), rowc(3), rowc(4), xt[:, 5, 0:8]], axis=1)
    p1 = jnp.concatenate(
        [xt[:, 5, 8:16], z8, rowc(6), rowc(7), rowc(8), rowc(9),
         xt[:, 10, 0:16]], axis=1)
    p2 = jnp.concatenate(
        [z8, rowc(11), rowc(12), rowc(13), rowc(14), rowc(15)], axis=1)
    v = jnp.stack([p0, p1, p2], axis=1)
    out_ref[...] = v.reshape(3 * BB // 16, 128)


def _build_table(us, prev_col):
    return pl.pallas_call(
        _build_body,
        grid=(NB,),
        in_specs=[
            pl.BlockSpec((C, BB), lambda i: (0, i)),
            pl.BlockSpec((C, NB), lambda i: (0, 0)),
        ],
        out_specs=pl.BlockSpec((3 * BB // 16, 128), lambda i: (i, 0)),
        out_shape=jax.ShapeDtypeStruct((3 * T // 16, 128), jnp.float32),
    )(us, prev_col)


def _sc_body(p_hbm, t_hbm, out_hbm,
             t_v0, t_v1, off_v0, off_v1, idx_v0, idx_v1,
             rows_v0, rows_v1, out_v0, out_v1,
             sem_t, sem_g0, sem_g1):
    wid = lax.axis_index("s") * NC + lax.axis_index("c")
    w_base = wid * QPW
    step = jnp.float32(STEP)
    scale = jnp.float32(SCALE)

    t_bufs = (t_v0, t_v1)
    off_bufs = (off_v0, off_v1)
    idx_bufs = (idx_v0, idx_v1)
    rows_bufs = (rows_v0, rows_v1)
    out_bufs = (out_v0, out_v1)
    sem_g = (sem_g0, sem_g1)

    def phase1(t_v, off_v, idx_v):
        for g in range(NGROUP):
            def vbody(i, carry2, g=g):
                q0 = g * GROUP + i * L
                t16 = t_v[pl.ds(q0, L)]
                f = t16 * scale
                j = f.astype(jnp.int32)
                jf = j.astype(jnp.float32)
                one = jnp.full((L,), 1, jnp.int32)
                zero = jnp.full((L,), 0, jnp.int32)
                a = jnp.where(jf * step < t16, one, zero)
                b = jnp.where((jf + 1.0) * step < t16, one, zero)
                idx = j + a + b - 1
                off_v[pl.ds(q0, L)] = t16 - idx.astype(jnp.float32) * step
                idx_v[g, pl.ds(i * L, L)] = idx
                return carry2

            lax.fori_loop(0, NVPG, vbody, 0, unroll=4)

    def fire_gathers(b):
        for g in range(NGROUP):
            pltpu.async_copy(
                p_hbm.at[idx_bufs[b].at[g]],
                rows_bufs[b].at[pl.ds(g * GROUP, GROUP)],
                sem_g[b],
            )

    def drain_gathers(b):
        for g in range(NGROUP):
            pltpu.make_async_copy(
                p_hbm.at[idx_bufs[b].at[g]],
                rows_bufs[b].at[pl.ds(g * GROUP, GROUP)],
                sem_g[b],
            ).wait()

    def phase3_and_write(k, b):
        rows_v, off_v, out_v = rows_bufs[b], off_bufs[b], out_bufs[b]

        def obody(jj, carry2):
            def sbody(s, carry3):
                q0 = jj * 128 + s * L
                off16 = off_v[pl.ds(q0, L)]
                qi = q0 + lax.iota(jnp.int32, L)
                for c in range(C):
                    ci = jnp.full((L,), c, jnp.int32)
                    di = jnp.full((L,), c + C, jnp.int32)
                    usc = plsc.load_gather(rows_v, [qi, ci])
                    dc = plsc.load_gather(rows_v, [qi, di])
                    out_v[jj, c, pl.ds(s * L, L)] = usc + dc * off16
                return carry3

            return lax.fori_loop(0, 128 // L, sbody, carry2, unroll=2)

        lax.fori_loop(0, CHUNK // 128, obody, 0)
        base = w_base + k * CHUNK
        pltpu.sync_copy(out_v, out_hbm.at[pl.ds(base // 128, CHUNK // 128)])

    def start_tload(k, b):
        pltpu.async_copy(
            t_hbm.at[pl.ds(w_base + k * CHUNK, CHUNK)], t_bufs[b], sem_t)

    def wait_tload(b):
        pltpu.make_async_copy(
            t_hbm.at[pl.ds(w_base, CHUNK)], t_bufs[b], sem_t).wait()

    def half(kk, parity):
        k_new = 2 * kk + (1 if parity == 0 else 2)
        bn = 1 if parity == 0 else 0
        br = 1 - bn

        @pl.when(k_new < NCHUNK)
        def _():
            wait_tload(bn)
            phase1(t_bufs[bn], off_bufs[bn], idx_bufs[bn])
            fire_gathers(bn)

        @pl.when(k_new + 1 < NCHUNK)
        def _():
            start_tload(k_new + 1, br)

        drain_gathers(br)
        phase3_and_write(k_new - 1, br)

    pltpu.sync_copy(t_hbm.at[pl.ds(w_base, CHUNK)], t_v0)
    phase1(t_v0, off_v0, idx_v0)
    fire_gathers(0)
    start_tload(1, 1)

    def pair_body(kk, carry):
        half(kk, 0)
        half(kk, 1)
        return carry

    lax.fori_loop(0, NCHUNK // 2, pair_body, 0)


def _sc_lookup(p_table, t):
    mesh = plsc.VectorSubcoreMesh(core_axis_name="c", subcore_axis_name="s")
    fn = functools.partial(
        pl.kernel,
        mesh=mesh,
        out_type=jax.ShapeDtypeStruct((N // 128, C, 128), jnp.float32),
        scratch_types=[
            pltpu.VMEM((CHUNK,), jnp.float32),
            pltpu.VMEM((CHUNK,), jnp.float32),
            pltpu.VMEM((CHUNK,), jnp.float32),
            pltpu.VMEM((CHUNK,), jnp.float32),
            pltpu.VMEM((NGROUP, GROUP), jnp.int32),
            pltpu.VMEM((NGROUP, GROUP), jnp.int32),
            pltpu.VMEM((CHUNK, RW), jnp.float32),
            pltpu.VMEM((CHUNK, RW), jnp.float32),
            pltpu.VMEM((CHUNK // 128, C, 128), jnp.float32),
            pltpu.VMEM((CHUNK // 128, C, 128), jnp.float32),
            pltpu.SemaphoreType.DMA,
            pltpu.SemaphoreType.DMA,
            pltpu.SemaphoreType.DMA,
        ],
        compiler_params=pltpu.CompilerParams(
            use_tc_tiling_on_sc=False, needs_layout_passes=False),
    )(_sc_body)
    return fn(p_table, t)


KB = 128


def _format_body(a_ref, out_ref):
    for l in range(KB):
        out_ref[:, pl.ds(l * 128, 128)] = a_ref[l]


def _format(a):
    return pl.pallas_call(
        _format_body,
        grid=(N // (128 * KB),),
        in_specs=[pl.BlockSpec((KB, C, 128), lambda i: (i, 0, 0))],
        out_specs=pl.BlockSpec((C, KB * 128), lambda i: (0, i)),
        out_shape=jax.ShapeDtypeStruct((C, N), jnp.float32),
    )(a)


def kernel(x, t, us, t_range):
    del x, t_range
    prev_col = jnp.concatenate(
        [us[:, :1], us[:, BB - 1 : T - 1 : BB]], axis=1)
    p_table = _build_table(us, prev_col).reshape(T, RW)
    return _format(_sc_lookup(p_table, t))


# speedup vs baseline: 1.1398x; 1.0658x over previous
import functools

import jax
import jax.numpy as jnp
import numpy as np
from jax import lax
from jax.experimental import pallas as pl
from jax.experimental.pallas import tpu as pltpu
from jax.experimental.pallas import tpu_sc as plsc

C = 8
T = 262144
N = 2097152

STEP = float(np.float32(1.0) / np.float32(T - 1))
SCALE = float(np.float32(T - 1))

RW = 24
NC, NS, L = 2, 16, 16
NW = NC * NS
CHUNK = 1024
GROUP = 128
NGROUP = CHUNK // GROUP
NVPG = GROUP // L
QPW = N // NW
NCHUNK = QPW // CHUNK


BB = 4096
NB = T // BB


def _build_body(us_ref, prev_ref, out_ref):
    ub = us_ref[...]
    pid = pl.program_id(0)
    pall = prev_ref[...]
    sel = lax.broadcasted_iota(jnp.int32, (1, NB), 1) == pid
    pc = jnp.sum(jnp.where(sel, pall, jnp.float32(0.0)), axis=1,
                 keepdims=True)
    ubp = jnp.concatenate([pc, ub[:, :-1]], axis=1)
    jint = lax.broadcasted_iota(jnp.int32, (1, BB), 1) + pid * BB
    jf = jint.astype(jnp.float32)
    step = jnp.float32(STEP)
    dx = jf * step - (jf - 1.0) * step
    d = (ub - ubp) / dx
    d = jnp.where(jint >= 2, d, jnp.float32(0.0))
    x = jnp.concatenate([ub, d], axis=0)
    xt = x.T.reshape(BB // 16, 16, 2 * C)
    z8 = jnp.zeros((BB // 16, 8), jnp.float32)

    def rowc(r):
        return jnp.concatenate([xt[:, r, :], z8], axis=1)

    p0 = jnp.concatenate(
        [rowc(0), rowc(1), rowc(2), rowc(3), rowc(4), xt[:, 5, 0:8]], axis=1)
    p1 = jnp.concatenate(
        [xt[:, 5, 8:16], z8, rowc(6), rowc(7), rowc(8), rowc(9),
         xt[:, 10, 0:16]], axis=1)
    p2 = jnp.concatenate(
        [z8, rowc(11), rowc(12), rowc(13), rowc(14), rowc(15)], axis=1)
    v = jnp.stack([p0, p1, p2], axis=1)
    out_ref[...] = v.reshape(3 * BB // 16, 128)


def _build_table(us, prev_col):
    return pl.pallas_call(
        _build_body,
        grid=(NB,),
        in_specs=[
            pl.BlockSpec((C, BB), lambda i: (0, i)),
            pl.BlockSpec((C, NB), lambda i: (0, 0)),
        ],
        out_specs=pl.BlockSpec((3 * BB // 16, 128), lambda i: (i, 0)),
        out_shape=jax.ShapeDtypeStruct((3 * T // 16, 128), jnp.float32),
    )(us, prev_col)


def _sc_body(p_hbm, t_hbm, out_hbm,
             t_v0, t_v1, off_v0, off_v1, idx_v0, idx_v1,
             rows_v0, rows_v1, out_v0, out_v1,
             sem_t, sem_g0, sem_g1):
    wid = lax.axis_index("s") * NC + lax.axis_index("c")
    w_base = wid * QPW
    step = jnp.float32(STEP)
    scale = jnp.float32(SCALE)

    t_bufs = (t_v0, t_v1)
    off_bufs = (off_v0, off_v1)
    idx_bufs = (idx_v0, idx_v1)
    rows_bufs = (rows_v0, rows_v1)
    out_bufs = (out_v0, out_v1)
    sem_g = (sem_g0, sem_g1)

    def phase1(t_v, off_v, idx_v):
        for g in range(NGROUP):
            def vbody(i, carry2, g=g):
                q0 = g * GROUP + i * L
                t16 = t_v[pl.ds(q0, L)]
                f = t16 * scale
                j = f.astype(jnp.int32)
                jf = j.astype(jnp.float32)
                one = jnp.full((L,), 1, jnp.int32)
                zero = jnp.full((L,), 0, jnp.int32)
                a = jnp.where(jf * step < t16, one, zero)
                b = jnp.where((jf + 1.0) * step < t16, one, zero)
                idx = j + a + b - 1
                off_v[pl.ds(q0, L)] = t16 - idx.astype(jnp.float32) * step
                idx_v[g, pl.ds(i * L, L)] = idx
                return carry2

            lax.fori_loop(0, NVPG, vbody, 0, unroll=4)

    def fire_gathers(b):
        for g in range(NGROUP):
            pltpu.async_copy(
                p_hbm.at[idx_bufs[b].at[g]],
                rows_bufs[b].at[pl.ds(g * GROUP, GROUP)],
                sem_g[b],
            )

    def drain_gathers(b):
        for g in range(NGROUP):
            pltpu.make_async_copy(
                p_hbm.at[idx_bufs[b].at[g]],
                rows_bufs[b].at[pl.ds(g * GROUP, GROUP)],
                sem_g[b],
            ).wait()

    def phase3_and_write(k, b):
        rows_v, off_v, out_v = rows_bufs[b], off_bufs[b], out_bufs[b]

        def obody(jj, carry2):
            def sbody(s, carry3):
                q0 = jj * 128 + s * L
                off16 = off_v[pl.ds(q0, L)]
                qi = q0 + lax.iota(jnp.int32, L)
                for c in range(C):
                    ci = jnp.full((L,), c, jnp.int32)
                    di = jnp.full((L,), c + C, jnp.int32)
                    usc = plsc.load_gather(rows_v, [qi, ci])
                    dc = plsc.load_gather(rows_v, [qi, di])
                    out_v[jj, c, pl.ds(s * L, L)] = usc + dc * off16
                return carry3

            return lax.fori_loop(0, 128 // L, sbody, carry2, unroll=2)

        lax.fori_loop(0, CHUNK // 128, obody, 0)
        base = w_base + k * CHUNK
        pltpu.sync_copy(out_v, out_hbm.at[pl.ds(base // 128, CHUNK // 128)])

    def start_tload(k, b):
        pltpu.async_copy(
            t_hbm.at[pl.ds(w_base + k * CHUNK, CHUNK)], t_bufs[b], sem_t)

    def wait_tload(b):
        pltpu.make_async_copy(
            t_hbm.at[pl.ds(w_base, CHUNK)], t_bufs[b], sem_t).wait()

    def half(kk, parity):
        k_new = 2 * kk + (1 if parity == 0 else 2)
        bn = 1 if parity == 0 else 0
        br = 1 - bn

        @pl.when(k_new < NCHUNK)
        def _():
            wait_tload(bn)
            phase1(t_bufs[bn], off_bufs[bn], idx_bufs[bn])
            fire_gathers(bn)

        @pl.when(k_new + 1 < NCHUNK)
        def _():
            start_tload(k_new + 1, br)

        drain_gathers(br)
        phase3_and_write(k_new - 1, br)

    pltpu.sync_copy(t_hbm.at[pl.ds(w_base, CHUNK)], t_v0)
    phase1(t_v0, off_v0, idx_v0)
    fire_gathers(0)
    start_tload(1, 1)

    def pair_body(kk, carry):
        half(kk, 0)
        half(kk, 1)
        return carry

    lax.fori_loop(0, NCHUNK // 2, pair_body, 0)


def _sc_lookup(p_table, t):
    mesh = plsc.VectorSubcoreMesh(core_axis_name="c", subcore_axis_name="s")
    fn = functools.partial(
        pl.kernel,
        mesh=mesh,
        out_type=jax.ShapeDtypeStruct((N // 128, C, 128), jnp.float32),
        scratch_types=[
            pltpu.VMEM((CHUNK,), jnp.float32),
            pltpu.VMEM((CHUNK,), jnp.float32),
            pltpu.VMEM((CHUNK,), jnp.float32),
            pltpu.VMEM((CHUNK,), jnp.float32),
            pltpu.VMEM((NGROUP, GROUP), jnp.int32),
            pltpu.VMEM((NGROUP, GROUP), jnp.int32),
            pltpu.VMEM((CHUNK, RW), jnp.float32),
            pltpu.VMEM((CHUNK, RW), jnp.float32),
            pltpu.VMEM((CHUNK // 128, C, 128), jnp.float32),
            pltpu.VMEM((CHUNK // 128, C, 128), jnp.float32),
            pltpu.SemaphoreType.DMA,
            pltpu.SemaphoreType.DMA,
            pltpu.SemaphoreType.DMA,
        ],
        compiler_params=pltpu.CompilerParams(
            use_tc_tiling_on_sc=False, needs_layout_passes=False),
    )(_sc_body)
    return fn(p_table, t)


KB = 256


def _format_body(a_ref, out_ref):
    for l in range(KB):
        out_ref[:, pl.ds(l * 128, 128)] = a_ref[l]


def _format(a):
    return pl.pallas_call(
        _format_body,
        grid=(N // (128 * KB),),
        in_specs=[pl.BlockSpec((KB, C, 128), lambda i: (i, 0, 0))],
        out_specs=pl.BlockSpec((C, KB * 128), lambda i: (0, i)),
        out_shape=jax.ShapeDtypeStruct((C, N), jnp.float32),
    )(a)


def kernel(x, t, us, t_range):
    del x, t_range
    prev_col = jnp.concatenate(
        [us[:, :1], us[:, BB - 1 : T - 1 : BB]], axis=1)
    p_table = _build_table(us, prev_col).reshape(T, RW)
    return _format(_sc_lookup(p_table, t))


# speedup vs baseline: 1.1783x; 1.0338x over previous
import functools

import jax
import jax.numpy as jnp
import numpy as np
from jax import lax
from jax.experimental import pallas as pl
from jax.experimental.pallas import tpu as pltpu
from jax.experimental.pallas import tpu_sc as plsc

C = 8
T = 262144
N = 2097152

STEP = float(np.float32(1.0) / np.float32(T - 1))
SCALE = float(np.float32(T - 1))

RW = 24
NC, NS, L = 2, 16, 16
NW = NC * NS
CHUNK = 1024
GROUP = 128
NGROUP = CHUNK // GROUP
NVPG = GROUP // L
QPW = N // NW
NCHUNK = QPW // CHUNK


BB = 4096
NB = T // BB


def _build_body(us_ref, prev_ref, out_ref):
    ub = us_ref[...]
    pid = pl.program_id(0)
    pall = prev_ref[...]
    sel = lax.broadcasted_iota(jnp.int32, (1, NB), 1) == pid
    pc = jnp.sum(jnp.where(sel, pall, jnp.float32(0.0)), axis=1,
                 keepdims=True)
    ubp = jnp.concatenate([pc, ub[:, :-1]], axis=1)
    jint = lax.broadcasted_iota(jnp.int32, (1, BB), 1) + pid * BB
    jf = jint.astype(jnp.float32)
    step = jnp.float32(STEP)
    dx = jf * step - (jf - 1.0) * step
    d = (ub - ubp) / dx
    d = jnp.where(jint >= 2, d, jnp.float32(0.0))
    x = jnp.concatenate([ub, d], axis=0)
    xt = x.T.reshape(BB // 16, 16, 2 * C)
    z8 = jnp.zeros((BB // 16, 8), jnp.float32)

    def rowc(r):
        return jnp.concatenate([xt[:, r, :], z8], axis=1)

    p0 = jnp.concatenate(
        [rowc(0), rowc(1), rowc(2), rowc(3), rowc(4), xt[:, 5, 0:8]], axis=1)
    p1 = jnp.concatenate(
        [xt[:, 5, 8:16], z8, rowc(6), rowc(7), rowc(8), rowc(9),
         xt[:, 10, 0:16]], axis=1)
    p2 = jnp.concatenate(
        [z8, rowc(11), rowc(12), rowc(13), rowc(14), rowc(15)], axis=1)
    v = jnp.stack([p0, p1, p2], axis=1)
    out_ref[...] = v.reshape(3 * BB // 16, 128)


def _build_table(us, prev_col):
    return pl.pallas_call(
        _build_body,
        grid=(NB,),
        in_specs=[
            pl.BlockSpec((C, BB), lambda i: (0, i)),
            pl.BlockSpec((C, NB), lambda i: (0, 0)),
        ],
        out_specs=pl.BlockSpec((3 * BB // 16, 128), lambda i: (i, 0)),
        out_shape=jax.ShapeDtypeStruct((3 * T // 16, 128), jnp.float32),
    )(us, prev_col)


def _sc_body(p_hbm, t_hbm, out_hbm,
             t_v0, t_v1, off_v0, off_v1, idx_v0, idx_v1,
             rows_v0, rows_v1, out_v0, out_v1,
             sem_t, sem_g0, sem_g1):
    wid = lax.axis_index("s") * NC + lax.axis_index("c")
    w_base = wid * QPW
    step = jnp.float32(STEP)
    scale = jnp.float32(SCALE)

    t_bufs = (t_v0, t_v1)
    off_bufs = (off_v0, off_v1)
    idx_bufs = (idx_v0, idx_v1)
    rows_bufs = (rows_v0, rows_v1)
    out_bufs = (out_v0, out_v1)
    sem_g = (sem_g0, sem_g1)

    def phase1(t_v, off_v, idx_v):
        for g in range(NGROUP):
            def vbody(i, carry2, g=g):
                q0 = g * GROUP + i * L
                t16 = t_v[pl.ds(q0, L)]
                f = t16 * scale
                j = f.astype(jnp.int32)
                jf = j.astype(jnp.float32)
                one = jnp.full((L,), 1, jnp.int32)
                zero = jnp.full((L,), 0, jnp.int32)
                a = jnp.where(jf * step < t16, one, zero)
                b = jnp.where((jf + 1.0) * step < t16, one, zero)
                idx = j + a + b - 1
                off_v[pl.ds(q0, L)] = t16 - idx.astype(jnp.float32) * step
                idx_v[g, pl.ds(i * L, L)] = idx
                return carry2

            lax.fori_loop(0, NVPG, vbody, 0, unroll=4)

    def fire_gathers(b):
        for g in range(NGROUP):
            pltpu.async_copy(
                p_hbm.at[idx_bufs[b].at[g]],
                rows_bufs[b].at[pl.ds(g * GROUP, GROUP)],
                sem_g[b],
            )

    def drain_gathers(b):
        for g in range(NGROUP):
            pltpu.make_async_copy(
                p_hbm.at[idx_bufs[b].at[g]],
                rows_bufs[b].at[pl.ds(g * GROUP, GROUP)],
                sem_g[b],
            ).wait()

    def phase3_and_write(k, b):
        rows_v, off_v, out_v = rows_bufs[b], off_bufs[b], out_bufs[b]

        def obody(jj, carry2):
            def sbody(s, carry3):
                q0 = jj * 128 + s * L
                off16 = off_v[pl.ds(q0, L)]
                qi = q0 + lax.iota(jnp.int32, L)
                for c in range(C):
                    ci = jnp.full((L,), c, jnp.int32)
                    di = jnp.full((L,), c + C, jnp.int32)
                    usc = plsc.load_gather(rows_v, [qi, ci])
                    dc = plsc.load_gather(rows_v, [qi, di])
                    out_v[jj, c, pl.ds(s * L, L)] = usc + dc * off16
                return carry3

            return lax.fori_loop(0, 128 // L, sbody, carry2, unroll=2)

        lax.fori_loop(0, CHUNK // 128, obody, 0)
        base = w_base + k * CHUNK
        pltpu.sync_copy(out_v, out_hbm.at[pl.ds(base // 128, CHUNK // 128)])

    def start_tload(k, b):
        pltpu.async_copy(
            t_hbm.at[pl.ds(w_base + k * CHUNK, CHUNK)], t_bufs[b], sem_t)

    def wait_tload(b):
        pltpu.make_async_copy(
            t_hbm.at[pl.ds(w_base, CHUNK)], t_bufs[b], sem_t).wait()

    def half(kk, parity):
        k_new = 2 * kk + (1 if parity == 0 else 2)
        bn = 1 if parity == 0 else 0
        br = 1 - bn

        @pl.when(k_new < NCHUNK)
        def _():
            wait_tload(bn)
            phase1(t_bufs[bn], off_bufs[bn], idx_bufs[bn])
            fire_gathers(bn)

        @pl.when(k_new + 1 < NCHUNK)
        def _():
            start_tload(k_new + 1, br)

        drain_gathers(br)
        phase3_and_write(k_new - 1, br)

    pltpu.sync_copy(t_hbm.at[pl.ds(w_base, CHUNK)], t_v0)
    phase1(t_v0, off_v0, idx_v0)
    fire_gathers(0)
    start_tload(1, 1)

    def pair_body(kk, carry):
        half(kk, 0)
        half(kk, 1)
        return carry

    lax.fori_loop(0, NCHUNK // 2, pair_body, 0)


def _sc_lookup(p_table, t):
    mesh = plsc.VectorSubcoreMesh(core_axis_name="c", subcore_axis_name="s")
    fn = functools.partial(
        pl.kernel,
        mesh=mesh,
        out_type=jax.ShapeDtypeStruct((N // 128, C, 128), jnp.float32),
        scratch_types=[
            pltpu.VMEM((CHUNK,), jnp.float32),
            pltpu.VMEM((CHUNK,), jnp.float32),
            pltpu.VMEM((CHUNK,), jnp.float32),
            pltpu.VMEM((CHUNK,), jnp.float32),
            pltpu.VMEM((NGROUP, GROUP), jnp.int32),
            pltpu.VMEM((NGROUP, GROUP), jnp.int32),
            pltpu.VMEM((CHUNK, RW), jnp.float32),
            pltpu.VMEM((CHUNK, RW), jnp.float32),
            pltpu.VMEM((CHUNK // 128, C, 128), jnp.float32),
            pltpu.VMEM((CHUNK // 128, C, 128), jnp.float32),
            pltpu.SemaphoreType.DMA,
            pltpu.SemaphoreType.DMA,
            pltpu.SemaphoreType.DMA,
        ],
        compiler_params=pltpu.CompilerParams(
            use_tc_tiling_on_sc=False, needs_layout_passes=False),
    )(_sc_body)
    return fn(p_table, t)


KB = 512


def _format_body(a_ref, out_ref):
    for l in range(KB):
        out_ref[:, pl.ds(l * 128, 128)] = a_ref[l]


def _format(a):
    return pl.pallas_call(
        _format_body,
        grid=(N // (128 * KB),),
        in_specs=[pl.BlockSpec((KB, C, 128), lambda i: (i, 0, 0))],
        out_specs=pl.BlockSpec((C, KB * 128), lambda i: (0, i)),
        out_shape=jax.ShapeDtypeStruct((C, N), jnp.float32),
    )(a)


def kernel(x, t, us, t_range):
    del x, t_range
    prev_col = jnp.concatenate(
        [us[:, :1], us[:, BB - 1 : T - 1 : BB]], axis=1)
    p_table = _build_table(us, prev_col).reshape(T, RW)
    return _format(_sc_lookup(p_table, t))


# speedup vs baseline: 1.1885x; 1.0086x over previous
import functools

import jax
import jax.numpy as jnp
import numpy as np
from jax import lax
from jax.experimental import pallas as pl
from jax.experimental.pallas import tpu as pltpu
from jax.experimental.pallas import tpu_sc as plsc

C = 8
T = 262144
N = 2097152

STEP = float(np.float32(1.0) / np.float32(T - 1))
SCALE = float(np.float32(T - 1))

RW = 24
NC, NS, L = 2, 16, 16
NW = NC * NS
CHUNK = 1024
GROUP = 128
NGROUP = CHUNK // GROUP
NVPG = GROUP // L
QPW = N // NW
NCHUNK = QPW // CHUNK


BB = 4096
NB = T // BB


def _build_body(us_ref, prev_ref, out_ref):
    ub = us_ref[...]
    pid = pl.program_id(0)
    pall = prev_ref[...]
    sel = lax.broadcasted_iota(jnp.int32, (1, NB), 1) == pid
    pc = jnp.sum(jnp.where(sel, pall, jnp.float32(0.0)), axis=1,
                 keepdims=True)
    ubp = jnp.concatenate([pc, ub[:, :-1]], axis=1)
    jint = lax.broadcasted_iota(jnp.int32, (1, BB), 1) + pid * BB
    jf = jint.astype(jnp.float32)
    step = jnp.float32(STEP)
    dx = jf * step - (jf - 1.0) * step
    d = (ub - ubp) / dx
    d = jnp.where(jint >= 2, d, jnp.float32(0.0))
    x = jnp.concatenate([ub, d], axis=0)
    xt = x.T.reshape(BB // 16, 16, 2 * C)
    z8 = jnp.zeros((BB // 16, 8), jnp.float32)

    def rowc(r):
        return jnp.concatenate([xt[:, r, :], z8], axis=1)

    p0 = jnp.concatenate(
        [rowc(0), rowc(1), rowc(2), rowc(3), rowc(4), xt[:, 5, 0:8]], axis=1)
    p1 = jnp.concatenate(
        [xt[:, 5, 8:16], z8, rowc(6), rowc(7), rowc(8), rowc(9),
         xt[:, 10, 0:16]], axis=1)
    p2 = jnp.concatenate(
        [z8, rowc(11), rowc(12), rowc(13), rowc(14), rowc(15)], axis=1)
    v = jnp.stack([p0, p1, p2], axis=1)
    out_ref[...] = v.reshape(3 * BB // 16, 128)


def _build_table(us, prev_col):
    return pl.pallas_call(
        _build_body,
        grid=(NB,),
        in_specs=[
            pl.BlockSpec((C, BB), lambda i: (0, i)),
            pl.BlockSpec((C, NB), lambda i: (0, 0)),
        ],
        out_specs=pl.BlockSpec((3 * BB // 16, 128), lambda i: (i, 0)),
        out_shape=jax.ShapeDtypeStruct((3 * T // 16, 128), jnp.float32),
    )(us, prev_col)


def _sc_body(p_hbm, t_hbm, out_hbm,
             t_v0, t_v1, off_v0, off_v1, idx_v0, idx_v1,
             rows_v0, rows_v1, out_v0, out_v1,
             sem_t, sem_g0, sem_g1):
    wid = lax.axis_index("s") * NC + lax.axis_index("c")
    w_base = wid * QPW
    step = jnp.float32(STEP)
    scale = jnp.float32(SCALE)

    t_bufs = (t_v0, t_v1)
    off_bufs = (off_v0, off_v1)
    idx_bufs = (idx_v0, idx_v1)
    rows_bufs = (rows_v0, rows_v1)
    out_bufs = (out_v0, out_v1)
    sem_g = (sem_g0, sem_g1)

    def phase1(t_v, off_v, idx_v):
        for g in range(NGROUP):
            def vbody(i, carry2, g=g):
                q0 = g * GROUP + i * L
                t16 = t_v[pl.ds(q0, L)]
                f = t16 * scale
                j = f.astype(jnp.int32)
                jf = j.astype(jnp.float32)
                one = jnp.full((L,), 1, jnp.int32)
                zero = jnp.full((L,), 0, jnp.int32)
                a = jnp.where(jf * step < t16, one, zero)
                b = jnp.where((jf + 1.0) * step < t16, one, zero)
                idx = j + a + b - 1
                off_v[pl.ds(q0, L)] = t16 - idx.astype(jnp.float32) * step
                idx_v[g, pl.ds(i * L, L)] = idx
                return carry2

            lax.fori_loop(0, NVPG, vbody, 0, unroll=4)

    def fire_gathers(b):
        for g in range(NGROUP):
            pltpu.async_copy(
                p_hbm.at[idx_bufs[b].at[g]],
                rows_bufs[b].at[pl.ds(g * GROUP, GROUP)],
                sem_g[b],
            )

    def drain_gathers(b):
        for g in range(NGROUP):
            pltpu.make_async_copy(
                p_hbm.at[idx_bufs[b].at[g]],
                rows_bufs[b].at[pl.ds(g * GROUP, GROUP)],
                sem_g[b],
            ).wait()

    def phase3_and_write(k, b):
        rows_v, off_v, out_v = rows_bufs[b], off_bufs[b], out_bufs[b]

        def obody(jj, carry2):
            def sbody(s, carry3):
                q0 = jj * 128 + s * L
                off16 = off_v[pl.ds(q0, L)]
                qi = q0 + lax.iota(jnp.int32, L)
                for c in range(C):
                    ci = jnp.full((L,), c, jnp.int32)
                    di = jnp.full((L,), c + C, jnp.int32)
                    usc = plsc.load_gather(rows_v, [qi, ci])
                    dc = plsc.load_gather(rows_v, [qi, di])
                    out_v[jj, c, pl.ds(s * L, L)] = usc + dc * off16
                return carry3

            return lax.fori_loop(0, 128 // L, sbody, carry2, unroll=2)

        lax.fori_loop(0, CHUNK // 128, obody, 0)
        base = w_base + k * CHUNK
        pltpu.sync_copy(out_v, out_hbm.at[pl.ds(base // 128, CHUNK // 128)])

    def start_tload(k, b):
        pltpu.async_copy(
            t_hbm.at[pl.ds(w_base + k * CHUNK, CHUNK)], t_bufs[b], sem_t)

    def wait_tload(b):
        pltpu.make_async_copy(
            t_hbm.at[pl.ds(w_base, CHUNK)], t_bufs[b], sem_t).wait()

    def half(kk, parity):
        k_new = 2 * kk + (1 if parity == 0 else 2)
        bn = 1 if parity == 0 else 0
        br = 1 - bn

        @pl.when(k_new < NCHUNK)
        def _():
            wait_tload(bn)
            phase1(t_bufs[bn], off_bufs[bn], idx_bufs[bn])
            fire_gathers(bn)

        @pl.when(k_new + 1 < NCHUNK)
        def _():
            start_tload(k_new + 1, br)

        drain_gathers(br)
        phase3_and_write(k_new - 1, br)

    pltpu.sync_copy(t_hbm.at[pl.ds(w_base, CHUNK)], t_v0)
    phase1(t_v0, off_v0, idx_v0)
    fire_gathers(0)
    start_tload(1, 1)

    def pair_body(kk, carry):
        half(kk, 0)
        half(kk, 1)
        return carry

    lax.fori_loop(0, NCHUNK // 2, pair_body, 0)


def _sc_lookup(p_table, t):
    mesh = plsc.VectorSubcoreMesh(core_axis_name="c", subcore_axis_name="s")
    fn = functools.partial(
        pl.kernel,
        mesh=mesh,
        out_type=jax.ShapeDtypeStruct((N // 128, C, 128), jnp.float32),
        scratch_types=[
            pltpu.VMEM((CHUNK,), jnp.float32),
            pltpu.VMEM((CHUNK,), jnp.float32),
            pltpu.VMEM((CHUNK,), jnp.float32),
            pltpu.VMEM((CHUNK,), jnp.float32),
            pltpu.VMEM((NGROUP, GROUP), jnp.int32),
            pltpu.VMEM((NGROUP, GROUP), jnp.int32),
            pltpu.VMEM((CHUNK, RW), jnp.float32),
            pltpu.VMEM((CHUNK, RW), jnp.float32),
            pltpu.VMEM((CHUNK // 128, C, 128), jnp.float32),
            pltpu.VMEM((CHUNK // 128, C, 128), jnp.float32),
            pltpu.SemaphoreType.DMA,
            pltpu.SemaphoreType.DMA,
            pltpu.SemaphoreType.DMA,
        ],
        compiler_params=pltpu.CompilerParams(
            use_tc_tiling_on_sc=False, needs_layout_passes=False),
    )(_sc_body)
    return fn(p_table, t)


KB = 1024


def _format_body(a_ref, out_ref):
    for l in range(KB):
        out_ref[:, pl.ds(l * 128, 128)] = a_ref[l]


def _format(a):
    return pl.pallas_call(
        _format_body,
        grid=(N // (128 * KB),),
        in_specs=[pl.BlockSpec((KB, C, 128), lambda i: (i, 0, 0))],
        out_specs=pl.BlockSpec((C, KB * 128), lambda i: (0, i)),
        out_shape=jax.ShapeDtypeStruct((C, N), jnp.float32),
    )(a)


def kernel(x, t, us, t_range):
    del x, t_range
    prev_col = jnp.concatenate(
        [us[:, :1], us[:, BB - 1 : T - 1 : BB]], axis=1)
    p_table = _build_table(us, prev_col).reshape(T, RW)
    return _format(_sc_lookup(p_table, t))
